# Initial kernel scaffold; baseline (speedup 1.0000x reference)
import jax
import jax.numpy as jnp
from jax.experimental import pallas as pl


def kernel(x, edge_index, edge_pairs, Wl1, bl1, Wr1, Wl2, bl2, Wr2, Wlin, blin):
    raise NotImplementedError("write your pallas kernel here")


# speedup vs baseline: 3.7245x; 3.7245x over previous
import functools

import jax
import jax.numpy as jnp
from jax import lax
from jax.experimental import pallas as pl
from jax.experimental.pallas import tpu as pltpu
from jax.experimental.pallas import tpu_sc as plsc

N_NODES = 10000
N_EDGES = 320000
N_PAIRS = 100000
D = 128

NC = 2
NS = 16
NW = NC * NS
LANES = 16

NPAD = 10240
ROWS_PER_SUB = NPAD // NS
CH = 128
EPW = 10240
NCHUNK = EPW // CH

PPW = 3200
PCHUNK = PPW // CH

_MESH = plsc.VectorSubcoreMesh(
    core_axis_name="c", subcore_axis_name="s", num_cores=NC, num_subcores=NS)


@functools.partial(
    pl.kernel,
    out_type=[
        jax.ShapeDtypeStruct((NC, NPAD, D), jnp.float32),
        jax.ShapeDtypeStruct((NC, NPAD), jnp.float32),
    ],
    mesh=_MESH,
    scratch_types=[
        pltpu.VMEM((NCHUNK, CH), jnp.int32),
        pltpu.VMEM((NCHUNK, CH), jnp.int32),
        pltpu.VMEM((CH, D), jnp.float32),
        pltpu.VMEM((CH,), jnp.float32),
        pltpu.VMEM((ROWS_PER_SUB,), jnp.float32),
        pltpu.VMEM_SHARED((NPAD, D), jnp.float32),
        pltpu.VMEM_SHARED((NPAD,), jnp.float32),
        pltpu.SemaphoreType.DMA,
    ],
)
def _segsum(table, srcw, dstw, zrows, out, out_cnt,
            src_v, dst_v, rows_v, ones_v, zc_v, acc, acc_cnt, sem):
  c = lax.axis_index("c")
  s = lax.axis_index("s")
  w = c * NS + s
  pltpu.sync_copy(srcw.at[w], src_v)
  pltpu.sync_copy(dstw.at[w], dst_v)
  pltpu.sync_copy(zrows, rows_v)
  for k in range(ROWS_PER_SUB // CH):
    pltpu.sync_copy(rows_v, acc.at[pl.ds((s * (ROWS_PER_SUB // CH) + k) * CH, CH)])

  def zfill(i, carry):
    zc_v[pl.ds(i * LANES, LANES)] = jnp.zeros((LANES,), jnp.float32)
    ones_v[pl.ds(i * LANES, LANES)] = jnp.ones((LANES,), jnp.float32)
    return carry

  lax.fori_loop(0, CH // LANES, zfill, 0)

  def zfill2(i, carry):
    zc_v[pl.ds(CH + i * LANES, LANES)] = jnp.zeros((LANES,), jnp.float32)
    return carry

  lax.fori_loop(0, (ROWS_PER_SUB - CH) // LANES, zfill2, 0)
  pltpu.sync_copy(zc_v, acc_cnt.at[pl.ds(s * ROWS_PER_SUB, ROWS_PER_SUB)])
  plsc.subcore_barrier()

  def body(j, carry):
    pltpu.async_copy(table.at[src_v.at[j]], rows_v, sem).wait()
    pltpu.sync_copy(rows_v, acc.at[dst_v.at[j]], add=True)
    pltpu.sync_copy(ones_v, acc_cnt.at[dst_v.at[j]], add=True)
    return carry

  lax.fori_loop(0, NCHUNK, body, 0)
  plsc.subcore_barrier()
  pltpu.sync_copy(acc_cnt.at[pl.ds(s * ROWS_PER_SUB, ROWS_PER_SUB)],
                  out_cnt.at[c, pl.ds(s * ROWS_PER_SUB, ROWS_PER_SUB)])
  pltpu.sync_copy(acc.at[pl.ds(s * ROWS_PER_SUB, ROWS_PER_SUB)],
                  out.at[c, pl.ds(s * ROWS_PER_SUB, ROWS_PER_SUB)])


def _tc1_body(acc0, acc1, cnt0, cnt1, x, wl1t, bl1, wr1t, h_ref, cnt_ref):
  agg = acc0[0] + acc1[0]
  cnt = jnp.maximum(cnt0[0] + cnt1[0], 1.0)
  mean = agg / cnt
  h = mean @ wl1t[...] + bl1[...] + x[...] @ wr1t[...]
  h_ref[...] = jnp.maximum(h, 0.0)
  cnt_ref[...] = cnt


def _tc2_body(acc0, acc1, cnt, h, wl2t, bl2, wr2t, wdec, blinv, uv_ref):
  agg = acc0[0] + acc1[0]
  mean = agg / cnt[...]
  z = mean @ wl2t[...] + bl2[...] + h[...] @ wr2t[...]
  uv_ref[...] = z @ wdec[...] + blinv[...]


@functools.partial(
    pl.kernel,
    out_type=jax.ShapeDtypeStruct((NW, PPW), jnp.float32),
    mesh=_MESH,
    scratch_types=[
        pltpu.VMEM((PCHUNK, CH), jnp.int32),
        pltpu.VMEM((PCHUNK, CH), jnp.int32),
        pltpu.VMEM((CH,), jnp.float32),
        pltpu.VMEM((CH,), jnp.float32),
        pltpu.VMEM((PPW,), jnp.float32),
        pltpu.SemaphoreType.DMA,
        pltpu.SemaphoreType.DMA,
    ],
)
def _decode(u_hbm, v_hbm, spw, dpw, out, sp_v, dp_v, us_v, vs_v, o_v, semu, semv):
  c = lax.axis_index("c")
  s = lax.axis_index("s")
  w = c * NS + s
  pltpu.sync_copy(spw.at[w], sp_v)
  pltpu.sync_copy(dpw.at[w], dp_v)

  def body(j, carry):
    cu = pltpu.async_copy(u_hbm.at[sp_v.at[j]], us_v, semu)
    cv = pltpu.async_copy(v_hbm.at[dp_v.at[j]], vs_v, semv)
    cu.wait()
    cv.wait()

    def sig(k, carry2):
      sl = pl.ds(k * LANES, LANES)
      logit = us_v[sl] + vs_v[sl]
      o_v[pl.ds(j * CH + k * LANES, LANES)] = 1.0 / (1.0 + jnp.exp(-logit))
      return carry2

    lax.fori_loop(0, CH // LANES, sig, 0)
    return carry

  lax.fori_loop(0, PCHUNK, body, 0)
  pltpu.sync_copy(o_v, out.at[w])


def _pad_worker_idx(idx, n_real, n_pad_total, pad_val):
  per_w = n_real // NW
  per_w_pad = n_pad_total // NW
  r = idx.reshape(NW, per_w)
  pad = jnp.full((NW, per_w_pad - per_w), pad_val, dtype=idx.dtype)
  return jnp.concatenate([r, pad], axis=1)


def kernel(x, edge_index, edge_pairs, Wl1, bl1, Wr1, Wl2, bl2, Wr2, Wlin, blin):
  f32 = jnp.float32
  src = _pad_worker_idx(edge_index[0], N_EDGES, EPW * NW, N_NODES)
  src = src.reshape(NW, NCHUNK, CH)
  dst = _pad_worker_idx(edge_index[1], N_EDGES, EPW * NW, N_NODES)
  dst = dst.reshape(NW, NCHUNK, CH)

  x_pad = jnp.concatenate([x, jnp.zeros((NPAD - N_NODES, D), f32)], axis=0)
  zrows = jnp.zeros((CH, D), f32)

  wl1t = Wl1.T
  wr1t = Wr1.T
  wl2t = Wl2.T
  wr2t = Wr2.T
  bl1r = bl1.reshape(1, D)
  bl2r = bl2.reshape(1, D)
  wdec = jnp.zeros((D, D), f32)
  wdec = wdec.at[:, 0].set(Wlin[0, :D]).at[:, 1].set(Wlin[0, D:])
  blinv = jnp.zeros((1, D), f32).at[0, 0].set(blin[0])

  sp = _pad_worker_idx(edge_pairs[0], N_PAIRS, PPW * NW, 0).reshape(NW, PCHUNK, CH)
  dp = _pad_worker_idx(edge_pairs[1], N_PAIRS, PPW * NW, 0).reshape(NW, PCHUNK, CH)

  acc1, cnt1 = _segsum(x_pad, src, dst, zrows)
  cnt1r = cnt1.reshape(NC, NPAD, 1)

  BLK = 1024
  grid = (NPAD // BLK,)
  full = lambda shape: pl.BlockSpec(shape, lambda i: tuple(0 for _ in shape))
  h, cntb = pl.pallas_call(
      _tc1_body,
      grid=grid,
      in_specs=[
          pl.BlockSpec((1, BLK, D), lambda i: (0, i, 0)),
          pl.BlockSpec((1, BLK, D), lambda i: (1, i, 0)),
          pl.BlockSpec((1, BLK, 1), lambda i: (0, i, 0)),
          pl.BlockSpec((1, BLK, 1), lambda i: (1, i, 0)),
          pl.BlockSpec((BLK, D), lambda i: (i, 0)),
          full((D, D)), full((1, D)), full((D, D)),
      ],
      out_specs=[
          pl.BlockSpec((BLK, D), lambda i: (i, 0)),
          pl.BlockSpec((BLK, 1), lambda i: (i, 0)),
      ],
      out_shape=[
          jax.ShapeDtypeStruct((NPAD, D), f32),
          jax.ShapeDtypeStruct((NPAD, 1), f32),
      ],
  )(acc1, acc1, cnt1r, cnt1r, x_pad, wl1t, bl1r, wr1t)

  acc2, _ = _segsum(h, src, dst, zrows)

  uv = pl.pallas_call(
      _tc2_body,
      grid=grid,
      in_specs=[
          pl.BlockSpec((1, BLK, D), lambda i: (0, i, 0)),
          pl.BlockSpec((1, BLK, D), lambda i: (1, i, 0)),
          pl.BlockSpec((BLK, 1), lambda i: (i, 0)),
          pl.BlockSpec((BLK, D), lambda i: (i, 0)),
          full((D, D)), full((1, D)), full((D, D)), full((D, D)), full((1, D)),
      ],
      out_specs=pl.BlockSpec((BLK, D), lambda i: (i, 0)),
      out_shape=jax.ShapeDtypeStruct((NPAD, D), f32),
  )(acc2, acc2, cntb, h, wl2t, bl2r, wr2t, wdec, blinv)

  u = uv[:, 0]
  v = uv[:, 1]

  probs = _decode(u, v, sp, dp)
  return probs.reshape(NW, PPW)[:, :N_PAIRS // NW].reshape(-1)


# speedup vs baseline: 3.8324x; 1.0290x over previous
import functools

import jax
import jax.numpy as jnp
from jax import lax
from jax.experimental import pallas as pl
from jax.experimental.pallas import tpu as pltpu
from jax.experimental.pallas import tpu_sc as plsc

N_NODES = 10000
N_EDGES = 320000
N_PAIRS = 100000
D = 128

NC = 2
NS = 16
NW = NC * NS
LANES = 16

NPAD = 10240
ROWS_PER_SUB = NPAD // NS
CH = 128
EPW = 10240
NCHUNK = EPW // CH
WIN = 2

PPW = 3200
PCHUNK = PPW // CH

_MESH = plsc.VectorSubcoreMesh(
    core_axis_name="c", subcore_axis_name="s", num_cores=NC, num_subcores=NS)


@functools.partial(
    pl.kernel,
    out_type=[
        jax.ShapeDtypeStruct((NC, NPAD, D), jnp.float32),
        jax.ShapeDtypeStruct((NC, NPAD), jnp.float32),
    ],
    mesh=_MESH,
    scratch_types=[
        pltpu.VMEM((WIN, CH), jnp.int32),
        pltpu.VMEM((NCHUNK, CH), jnp.int32),
        pltpu.VMEM((WIN, CH, D), jnp.float32),
        pltpu.VMEM((CH,), jnp.float32),
        pltpu.VMEM((ROWS_PER_SUB,), jnp.float32),
        pltpu.VMEM_SHARED((NPAD, D), jnp.float32),
        pltpu.VMEM_SHARED((NPAD,), jnp.float32),
        [pltpu.SemaphoreType.DMA] * WIN,
        [pltpu.SemaphoreType.DMA] * WIN,
        pltpu.SemaphoreType.DMA,
    ],
)
def _segsum(table, srcw, dstw, zrows, out, out_cnt,
            src_v, dst_v, rows_v, ones_v, zc_v, acc, acc_cnt, gsems, ssems, csem):
  c = lax.axis_index("c")
  s = lax.axis_index("s")
  w = c * NS + s
  pltpu.sync_copy(dstw.at[w], dst_v)
  pltpu.sync_copy(zrows, rows_v.at[0])
  for k in range(ROWS_PER_SUB // CH):
    pltpu.sync_copy(rows_v.at[0], acc.at[pl.ds((s * (ROWS_PER_SUB // CH) + k) * CH, CH)])

  def zfill(i, carry):
    zc_v[pl.ds(i * LANES, LANES)] = jnp.zeros((LANES,), jnp.float32)
    ones_v[pl.ds(i * LANES, LANES)] = jnp.ones((LANES,), jnp.float32)
    return carry

  lax.fori_loop(0, CH // LANES, zfill, 0)

  def zfill2(i, carry):
    zc_v[pl.ds(CH + i * LANES, LANES)] = jnp.zeros((LANES,), jnp.float32)
    return carry

  lax.fori_loop(0, (ROWS_PER_SUB - CH) // LANES, zfill2, 0)
  pltpu.sync_copy(zc_v, acc_cnt.at[pl.ds(s * ROWS_PER_SUB, ROWS_PER_SUB)])
  plsc.subcore_barrier()

  def body(t, carry):
    pltpu.sync_copy(srcw.at[w, pl.ds(t * WIN, WIN)], src_v)
    gds = [
        pltpu.async_copy(table.at[src_v.at[b]], rows_v.at[b], gsems[b])
        for b in range(WIN)
    ]
    cds = [
        pltpu.async_copy(ones_v, acc_cnt.at[dst_v.at[t * WIN + b]], csem,
                         add=True)
        for b in range(WIN)
    ]
    sds = []
    for b in range(WIN):
      gds[b].wait()
      sds.append(pltpu.async_copy(rows_v.at[b], acc.at[dst_v.at[t * WIN + b]],
                                  ssems[b], add=True))
    for b in range(WIN):
      sds[b].wait()
      cds[b].wait()
    return carry

  lax.fori_loop(0, NCHUNK // WIN, body, 0)
  plsc.subcore_barrier()
  pltpu.sync_copy(acc_cnt.at[pl.ds(s * ROWS_PER_SUB, ROWS_PER_SUB)],
                  out_cnt.at[c, pl.ds(s * ROWS_PER_SUB, ROWS_PER_SUB)])
  pltpu.sync_copy(acc.at[pl.ds(s * ROWS_PER_SUB, ROWS_PER_SUB)],
                  out.at[c, pl.ds(s * ROWS_PER_SUB, ROWS_PER_SUB)])


def _tc1_body(acc0, acc1, cnt0, cnt1, x, wl1t, bl1, wr1t, h_ref, cnt_ref):
  agg = acc0[0] + acc1[0]
  cnt = jnp.maximum(cnt0[0] + cnt1[0], 1.0)
  mean = agg / cnt
  h = mean @ wl1t[...] + bl1[...] + x[...] @ wr1t[...]
  h_ref[...] = jnp.maximum(h, 0.0)
  cnt_ref[...] = cnt


def _tc2_body(acc0, acc1, cnt, h, wl2t, bl2, wr2t, wdec, blinv, uv_ref):
  agg = acc0[0] + acc1[0]
  mean = agg / cnt[...]
  z = mean @ wl2t[...] + bl2[...] + h[...] @ wr2t[...]
  uv_ref[...] = z @ wdec[...] + blinv[...]


@functools.partial(
    pl.kernel,
    out_type=jax.ShapeDtypeStruct((NW, PPW), jnp.float32),
    mesh=_MESH,
    scratch_types=[
        pltpu.VMEM((PCHUNK, CH), jnp.int32),
        pltpu.VMEM((PCHUNK, CH), jnp.int32),
        pltpu.VMEM((CH,), jnp.float32),
        pltpu.VMEM((CH,), jnp.float32),
        pltpu.VMEM((PPW,), jnp.float32),
        pltpu.SemaphoreType.DMA,
        pltpu.SemaphoreType.DMA,
    ],
)
def _decode(u_hbm, v_hbm, spw, dpw, out, sp_v, dp_v, us_v, vs_v, o_v, semu, semv):
  c = lax.axis_index("c")
  s = lax.axis_index("s")
  w = c * NS + s
  pltpu.sync_copy(spw.at[w], sp_v)
  pltpu.sync_copy(dpw.at[w], dp_v)

  def body(j, carry):
    cu = pltpu.async_copy(u_hbm.at[sp_v.at[j]], us_v, semu)
    cv = pltpu.async_copy(v_hbm.at[dp_v.at[j]], vs_v, semv)
    cu.wait()
    cv.wait()

    def sig(k, carry2):
      sl = pl.ds(k * LANES, LANES)
      logit = us_v[sl] + vs_v[sl]
      o_v[pl.ds(j * CH + k * LANES, LANES)] = 1.0 / (1.0 + jnp.exp(-logit))
      return carry2

    lax.fori_loop(0, CH // LANES, sig, 0)
    return carry

  lax.fori_loop(0, PCHUNK, body, 0)
  pltpu.sync_copy(o_v, out.at[w])


def _pad_worker_idx(idx, n_real, n_pad_total, pad_val):
  per_w = n_real // NW
  per_w_pad = n_pad_total // NW
  r = idx.reshape(NW, per_w)
  pad = jnp.full((NW, per_w_pad - per_w), pad_val, dtype=idx.dtype)
  return jnp.concatenate([r, pad], axis=1)


def kernel(x, edge_index, edge_pairs, Wl1, bl1, Wr1, Wl2, bl2, Wr2, Wlin, blin):
  f32 = jnp.float32
  src = _pad_worker_idx(edge_index[0], N_EDGES, EPW * NW, N_NODES)
  src = src.reshape(NW, NCHUNK, CH)
  dst = _pad_worker_idx(edge_index[1], N_EDGES, EPW * NW, N_NODES)
  dst = dst.reshape(NW, NCHUNK, CH)

  x_pad = jnp.concatenate([x, jnp.zeros((NPAD - N_NODES, D), f32)], axis=0)
  zrows = jnp.zeros((CH, D), f32)

  wl1t = Wl1.T
  wr1t = Wr1.T
  wl2t = Wl2.T
  wr2t = Wr2.T
  bl1r = bl1.reshape(1, D)
  bl2r = bl2.reshape(1, D)
  wdec = jnp.zeros((D, D), f32)
  wdec = wdec.at[:, 0].set(Wlin[0, :D]).at[:, 1].set(Wlin[0, D:])
  blinv = jnp.zeros((1, D), f32).at[0, 0].set(blin[0])

  sp = _pad_worker_idx(edge_pairs[0], N_PAIRS, PPW * NW, 0).reshape(NW, PCHUNK, CH)
  dp = _pad_worker_idx(edge_pairs[1], N_PAIRS, PPW * NW, 0).reshape(NW, PCHUNK, CH)

  acc1, cnt1 = _segsum(x_pad, src, dst, zrows)
  cnt1r = cnt1.reshape(NC, NPAD, 1)

  BLK = 1024
  grid = (NPAD // BLK,)
  full = lambda shape: pl.BlockSpec(shape, lambda i: tuple(0 for _ in shape))
  h, cntb = pl.pallas_call(
      _tc1_body,
      grid=grid,
      in_specs=[
          pl.BlockSpec((1, BLK, D), lambda i: (0, i, 0)),
          pl.BlockSpec((1, BLK, D), lambda i: (1, i, 0)),
          pl.BlockSpec((1, BLK, 1), lambda i: (0, i, 0)),
          pl.BlockSpec((1, BLK, 1), lambda i: (1, i, 0)),
          pl.BlockSpec((BLK, D), lambda i: (i, 0)),
          full((D, D)), full((1, D)), full((D, D)),
      ],
      out_specs=[
          pl.BlockSpec((BLK, D), lambda i: (i, 0)),
          pl.BlockSpec((BLK, 1), lambda i: (i, 0)),
      ],
      out_shape=[
          jax.ShapeDtypeStruct((NPAD, D), f32),
          jax.ShapeDtypeStruct((NPAD, 1), f32),
      ],
  )(acc1, acc1, cnt1r, cnt1r, x_pad, wl1t, bl1r, wr1t)

  acc2, _ = _segsum(h, src, dst, zrows)

  uv = pl.pallas_call(
      _tc2_body,
      grid=grid,
      in_specs=[
          pl.BlockSpec((1, BLK, D), lambda i: (0, i, 0)),
          pl.BlockSpec((1, BLK, D), lambda i: (1, i, 0)),
          pl.BlockSpec((BLK, 1), lambda i: (i, 0)),
          pl.BlockSpec((BLK, D), lambda i: (i, 0)),
          full((D, D)), full((1, D)), full((D, D)), full((D, D)), full((1, D)),
      ],
      out_specs=pl.BlockSpec((BLK, D), lambda i: (i, 0)),
      out_shape=jax.ShapeDtypeStruct((NPAD, D), f32),
  )(acc2, acc2, cntb, h, wl2t, bl2r, wr2t, wdec, blinv)

  u = uv[:, 0]
  v = uv[:, 1]

  probs = _decode(u, v, sp, dp)
  return probs.reshape(NW, PPW)[:, :N_PAIRS // NW].reshape(-1)


# speedup vs baseline: 4.1994x; 1.0958x over previous
import functools

import jax
import jax.numpy as jnp
from jax import lax
from jax.experimental import pallas as pl
from jax.experimental.pallas import tpu as pltpu
from jax.experimental.pallas import tpu_sc as plsc

N_NODES = 10000
N_EDGES = 320000
N_PAIRS = 100000
D = 128

NC = 2
NS = 16
NW = NC * NS
LANES = 16

NPAD = 10240
ROWS_PER_SUB = NPAD // NS
CH = 80
EPW = 10240
NCHUNK = EPW // CH
RB = 4
IB = 8
UNROLL = 8
NT = NCHUNK // UNROLL

DCH = 128
PPW = 3200
PCHUNK = PPW // DCH

_MESH = plsc.VectorSubcoreMesh(
    core_axis_name="c", subcore_axis_name="s", num_cores=NC, num_subcores=NS)


@functools.partial(
    pl.kernel,
    out_type=[
        jax.ShapeDtypeStruct((NC, NPAD, D), jnp.float32),
        jax.ShapeDtypeStruct((NC, NPAD), jnp.float32),
    ],
    mesh=_MESH,
    scratch_types=[
        pltpu.VMEM((IB, CH), jnp.int32),
        pltpu.VMEM((IB, CH), jnp.int32),
        pltpu.VMEM((RB, CH, D), jnp.float32),
        pltpu.VMEM((CH,), jnp.float32),
        pltpu.VMEM((DCH,), jnp.float32),
        pltpu.VMEM_SHARED((NPAD, D), jnp.float32),
        pltpu.VMEM_SHARED((NPAD,), jnp.float32),
        [pltpu.SemaphoreType.DMA] * IB,
        [pltpu.SemaphoreType.DMA] * RB,
        [pltpu.SemaphoreType.DMA] * RB,
        [pltpu.SemaphoreType.DMA] * RB,
    ],
)
def _segsum(table, srcw, dstw, zrows, out, out_cnt,
            src_v, dst_v, rows_v, ones_v, zc_v, acc, acc_cnt,
            isems, gsems, ssems, csems):
  c = lax.axis_index("c")
  s = lax.axis_index("s")
  w = c * NS + s
  pltpu.sync_copy(zrows, rows_v.at[0])
  for k in range(ROWS_PER_SUB // CH):
    pltpu.sync_copy(rows_v.at[0], acc.at[pl.ds((s * (ROWS_PER_SUB // CH) + k) * CH, CH)])

  def zfill(i, carry):
    zc_v[pl.ds(i * LANES, LANES)] = jnp.zeros((LANES,), jnp.float32)
    return carry

  lax.fori_loop(0, DCH // LANES, zfill, 0)

  def ofill(i, carry):
    ones_v[pl.ds(i * LANES, LANES)] = jnp.ones((LANES,), jnp.float32)
    return carry

  lax.fori_loop(0, CH // LANES, ofill, 0)
  for k in range(ROWS_PER_SUB // DCH):
    pltpu.sync_copy(zc_v, acc_cnt.at[pl.ds(s * ROWS_PER_SUB + k * DCH, DCH)])
  plsc.subcore_barrier()

  def wait_idx(qi):
    pltpu.make_async_copy(srcw.at[w, 0], src_v.at[qi], isems[qi]).wait()
    pltpu.make_async_copy(dstw.at[w, 0], dst_v.at[qi], isems[qi]).wait()

  def wait_gather(q):
    pltpu.make_async_copy(table.at[src_v.at[0]], rows_v.at[q], gsems[q]).wait()

  def wait_scatter(q):
    pltpu.make_async_copy(rows_v.at[q], acc.at[dst_v.at[0]], ssems[q]).wait()

  def wait_cnt(q):
    pltpu.make_async_copy(ones_v, acc_cnt.at[dst_v.at[0]], csems[q]).wait()

  for b in range(RB):
    pltpu.async_copy(srcw.at[w, b], src_v.at[b], isems[b])
    pltpu.async_copy(dstw.at[w, b], dst_v.at[b], isems[b])

  def outer(t, carry):
    for b in range(UNROLL):
      j = t * UNROLL + b
      q = b % RB
      qm = (b - 1) % RB
      qi = b % IB
      qim = (b - 1) % IB
      qp = (b + RB) % IB

      def waits_a():
        wait_scatter(q)
        wait_cnt(q)

      if b < RB:
        pl.when(t > 0)(waits_a)
      else:
        waits_a()

      def prefetch():
        pltpu.async_copy(srcw.at[w, j + RB], src_v.at[qp], isems[qp])
        pltpu.async_copy(dstw.at[w, j + RB], dst_v.at[qp], isems[qp])

      if b < RB:
        prefetch()
      else:
        pl.when(t < NT - 1)(prefetch)

      wait_idx(qi)
      pltpu.async_copy(table.at[src_v.at[qi]], rows_v.at[q], gsems[q])

      def scatter_prev():
        wait_gather(qm)
        pltpu.async_copy(rows_v.at[qm], acc.at[dst_v.at[qim]], ssems[qm],
                         add=True)
        pltpu.async_copy(ones_v, acc_cnt.at[dst_v.at[qim]], csems[qm],
                         add=True)

      if b == 0:
        pl.when(t > 0)(scatter_prev)
      else:
        scatter_prev()
    return carry

  lax.fori_loop(0, NT, outer, 0)
  qL = (NCHUNK - 1) % RB
  qiL = (NCHUNK - 1) % IB
  wait_gather(qL)
  pltpu.async_copy(rows_v.at[qL], acc.at[dst_v.at[qiL]], ssems[qL], add=True)
  pltpu.async_copy(ones_v, acc_cnt.at[dst_v.at[qiL]], csems[qL], add=True)
  for q in range(RB):
    wait_scatter(q)
    wait_cnt(q)
  plsc.subcore_barrier()
  pltpu.sync_copy(acc_cnt.at[pl.ds(s * ROWS_PER_SUB, ROWS_PER_SUB)],
                  out_cnt.at[c, pl.ds(s * ROWS_PER_SUB, ROWS_PER_SUB)])
  pltpu.sync_copy(acc.at[pl.ds(s * ROWS_PER_SUB, ROWS_PER_SUB)],
                  out.at[c, pl.ds(s * ROWS_PER_SUB, ROWS_PER_SUB)])


def _tc1_body(acc0, acc1, cnt0, cnt1, x, wl1t, bl1, wr1t, h_ref, cnt_ref):
  agg = acc0[0] + acc1[0]
  cnt = jnp.maximum(cnt0[0] + cnt1[0], 1.0)
  mean = agg / cnt
  h = mean @ wl1t[...] + bl1[...] + x[...] @ wr1t[...]
  h_ref[...] = jnp.maximum(h, 0.0)
  cnt_ref[...] = cnt


def _tc2_body(acc0, acc1, cnt, h, wl2t, bl2, wr2t, wdec, blinv, uv_ref):
  agg = acc0[0] + acc1[0]
  mean = agg / cnt[...]
  z = mean @ wl2t[...] + bl2[...] + h[...] @ wr2t[...]
  uv_ref[...] = z @ wdec[...] + blinv[...]


@functools.partial(
    pl.kernel,
    out_type=jax.ShapeDtypeStruct((NW, PPW), jnp.float32),
    mesh=_MESH,
    scratch_types=[
        pltpu.VMEM((PCHUNK, DCH), jnp.int32),
        pltpu.VMEM((PCHUNK, DCH), jnp.int32),
        pltpu.VMEM((DCH,), jnp.float32),
        pltpu.VMEM((DCH,), jnp.float32),
        pltpu.VMEM((PPW,), jnp.float32),
        pltpu.SemaphoreType.DMA,
        pltpu.SemaphoreType.DMA,
    ],
)
def _decode(u_hbm, v_hbm, spw, dpw, out, sp_v, dp_v, us_v, vs_v, o_v, semu, semv):
  c = lax.axis_index("c")
  s = lax.axis_index("s")
  w = c * NS + s
  pltpu.sync_copy(spw.at[w], sp_v)
  pltpu.sync_copy(dpw.at[w], dp_v)

  def body(j, carry):
    cu = pltpu.async_copy(u_hbm.at[sp_v.at[j]], us_v, semu)
    cv = pltpu.async_copy(v_hbm.at[dp_v.at[j]], vs_v, semv)
    cu.wait()
    cv.wait()

    def sig(k, carry2):
      sl = pl.ds(k * LANES, LANES)
      logit = us_v[sl] + vs_v[sl]
      o_v[pl.ds(j * DCH + k * LANES, LANES)] = 1.0 / (1.0 + jnp.exp(-logit))
      return carry2

    lax.fori_loop(0, DCH // LANES, sig, 0)
    return carry

  lax.fori_loop(0, PCHUNK, body, 0)
  pltpu.sync_copy(o_v, out.at[w])


def _pad_worker_idx(idx, n_real, n_pad_total, pad_val):
  per_w = n_real // NW
  per_w_pad = n_pad_total // NW
  r = idx.reshape(NW, per_w)
  pad = jnp.full((NW, per_w_pad - per_w), pad_val, dtype=idx.dtype)
  return jnp.concatenate([r, pad], axis=1)


def kernel(x, edge_index, edge_pairs, Wl1, bl1, Wr1, Wl2, bl2, Wr2, Wlin, blin):
  f32 = jnp.float32
  src = _pad_worker_idx(edge_index[0], N_EDGES, EPW * NW, N_NODES)
  src = src.reshape(NW, NCHUNK, CH)
  dst = _pad_worker_idx(edge_index[1], N_EDGES, EPW * NW, N_NODES)
  dst = dst.reshape(NW, NCHUNK, CH)

  x_pad = jnp.concatenate([x, jnp.zeros((NPAD - N_NODES, D), f32)], axis=0)
  zrows = jnp.zeros((CH, D), f32)

  wl1t = Wl1.T
  wr1t = Wr1.T
  wl2t = Wl2.T
  wr2t = Wr2.T
  bl1r = bl1.reshape(1, D)
  bl2r = bl2.reshape(1, D)
  wdec = jnp.zeros((D, D), f32)
  wdec = wdec.at[:, 0].set(Wlin[0, :D]).at[:, 1].set(Wlin[0, D:])
  blinv = jnp.zeros((1, D), f32).at[0, 0].set(blin[0])

  sp = _pad_worker_idx(edge_pairs[0], N_PAIRS, PPW * NW, 0).reshape(NW, PCHUNK, DCH)
  dp = _pad_worker_idx(edge_pairs[1], N_PAIRS, PPW * NW, 0).reshape(NW, PCHUNK, DCH)

  acc1, cnt1 = _segsum(x_pad, src, dst, zrows)
  cnt1r = cnt1.reshape(NC, NPAD, 1)

  BLK = 1024
  grid = (NPAD // BLK,)
  full = lambda shape: pl.BlockSpec(shape, lambda i: tuple(0 for _ in shape))
  h, cntb = pl.pallas_call(
      _tc1_body,
      grid=grid,
      in_specs=[
          pl.BlockSpec((1, BLK, D), lambda i: (0, i, 0)),
          pl.BlockSpec((1, BLK, D), lambda i: (1, i, 0)),
          pl.BlockSpec((1, BLK, 1), lambda i: (0, i, 0)),
          pl.BlockSpec((1, BLK, 1), lambda i: (1, i, 0)),
          pl.BlockSpec((BLK, D), lambda i: (i, 0)),
          full((D, D)), full((1, D)), full((D, D)),
      ],
      out_specs=[
          pl.BlockSpec((BLK, D), lambda i: (i, 0)),
          pl.BlockSpec((BLK, 1), lambda i: (i, 0)),
      ],
      out_shape=[
          jax.ShapeDtypeStruct((NPAD, D), f32),
          jax.ShapeDtypeStruct((NPAD, 1), f32),
      ],
  )(acc1, acc1, cnt1r, cnt1r, x_pad, wl1t, bl1r, wr1t)

  acc2, _ = _segsum(h, src, dst, zrows)

  uv = pl.pallas_call(
      _tc2_body,
      grid=grid,
      in_specs=[
          pl.BlockSpec((1, BLK, D), lambda i: (0, i, 0)),
          pl.BlockSpec((1, BLK, D), lambda i: (1, i, 0)),
          pl.BlockSpec((BLK, 1), lambda i: (i, 0)),
          pl.BlockSpec((BLK, D), lambda i: (i, 0)),
          full((D, D)), full((1, D)), full((D, D)), full((D, D)), full((1, D)),
      ],
      out_specs=pl.BlockSpec((BLK, D), lambda i: (i, 0)),
      out_shape=jax.ShapeDtypeStruct((NPAD, D), f32),
  )(acc2, acc2, cntb, h, wl2t, bl2r, wr2t, wdec, blinv)

  u = uv[:, 0]
  v = uv[:, 1]

  probs = _decode(u, v, sp, dp)
  return probs.reshape(NW, PPW)[:, :N_PAIRS // NW].reshape(-1)


# speedup vs baseline: 4.3267x; 1.0303x over previous
import functools

import jax
import jax.numpy as jnp
from jax import lax
from jax.experimental import pallas as pl
from jax.experimental.pallas import tpu as pltpu
from jax.experimental.pallas import tpu_sc as plsc

N_NODES = 10000
N_EDGES = 320000
N_PAIRS = 100000
D = 128

NC = 2
NS = 16
NW = NC * NS
LANES = 16

NPAD = 10240
ROWS_PER_SUB = NPAD // NS
CH = 64
EPW = 10240
NCHUNK = EPW // CH
RB = 5
SB = 2
IB = 10
UNROLL = 10
NT = NCHUNK // UNROLL
GA = 3
IA = 6

DCH = 128
PPW = 3200
PCHUNK = PPW // DCH

_MESH = plsc.VectorSubcoreMesh(
    core_axis_name="c", subcore_axis_name="s", num_cores=NC, num_subcores=NS)


@functools.partial(
    pl.kernel,
    out_type=[
        jax.ShapeDtypeStruct((NC, NPAD, D), jnp.float32),
        jax.ShapeDtypeStruct((NC, NPAD), jnp.float32),
    ],
    mesh=_MESH,
    scratch_types=[
        pltpu.VMEM((IB, CH), jnp.int32),
        pltpu.VMEM((IB, CH), jnp.int32),
        pltpu.VMEM((RB, CH, D), jnp.float32),
        pltpu.VMEM((CH,), jnp.float32),
        pltpu.VMEM((DCH,), jnp.float32),
        pltpu.VMEM_SHARED((NPAD, D), jnp.float32),
        pltpu.VMEM_SHARED((NPAD,), jnp.float32),
        [pltpu.SemaphoreType.DMA] * IB,
        [pltpu.SemaphoreType.DMA] * RB,
        [pltpu.SemaphoreType.DMA] * SB,
        [pltpu.SemaphoreType.DMA] * SB,
    ],
)
def _segsum(table, srcw, dstw, zrows, out, out_cnt,
            src_v, dst_v, rows_v, ones_v, zc_v, acc, acc_cnt,
            isems, gsems, ssems, csems):
  c = lax.axis_index("c")
  s = lax.axis_index("s")
  w = c * NS + s
  pltpu.sync_copy(zrows, rows_v.at[0])
  for k in range(ROWS_PER_SUB // CH):
    pltpu.sync_copy(rows_v.at[0], acc.at[pl.ds((s * (ROWS_PER_SUB // CH) + k) * CH, CH)])

  def zfill(i, carry):
    zc_v[pl.ds(i * LANES, LANES)] = jnp.zeros((LANES,), jnp.float32)
    return carry

  lax.fori_loop(0, DCH // LANES, zfill, 0)

  def ofill(i, carry):
    ones_v[pl.ds(i * LANES, LANES)] = jnp.ones((LANES,), jnp.float32)
    return carry

  lax.fori_loop(0, CH // LANES, ofill, 0)
  for k in range(ROWS_PER_SUB // DCH):
    pltpu.sync_copy(zc_v, acc_cnt.at[pl.ds(s * ROWS_PER_SUB + k * DCH, DCH)])
  plsc.subcore_barrier()

  def wait_idx(qi):
    pltpu.make_async_copy(srcw.at[w, 0], src_v.at[qi], isems[qi]).wait()
    pltpu.make_async_copy(dstw.at[w, 0], dst_v.at[qi], isems[qi]).wait()

  def wait_gather(q):
    pltpu.make_async_copy(table.at[src_v.at[0]], rows_v.at[q], gsems[q]).wait()

  def wait_scatter(p):
    pltpu.make_async_copy(rows_v.at[0], acc.at[dst_v.at[0]], ssems[p]).wait()

  def wait_cnt(p):
    pltpu.make_async_copy(ones_v, acc_cnt.at[dst_v.at[0]], csems[p]).wait()

  def prefetch_idx(jj, qslot):
    pltpu.async_copy(srcw.at[w, jj], src_v.at[qslot], isems[qslot])
    pltpu.async_copy(dstw.at[w, jj], dst_v.at[qslot], isems[qslot])

  for b in range(IA):
    prefetch_idx(b, b)
  for b in range(GA):
    wait_idx(b)
    pltpu.async_copy(table.at[src_v.at[b]], rows_v.at[b], gsems[b])

  def outer(t, carry):
    for b in range(UNROLL):
      j = t * UNROLL + b
      q = b % RB
      qg = (b + GA) % RB
      qi = b % IB
      qig = (b + GA) % IB
      qip = (b + IA) % IB
      p = b % SB

      def waits_a():
        wait_scatter(p)
        wait_cnt(p)

      if b < SB:
        pl.when(t > 0)(waits_a)
      else:
        waits_a()

      if b < IB - IA:
        prefetch_idx(j + IA, qip)
      else:
        pl.when(t < NT - 1)(lambda: prefetch_idx(j + IA, qip))

      wait_gather(q)
      pltpu.async_copy(rows_v.at[q], acc.at[dst_v.at[qi]], ssems[p], add=True)
      pltpu.async_copy(ones_v, acc_cnt.at[dst_v.at[qi]], csems[p], add=True)

      def gather_ahead():
        wait_idx(qig)
        pltpu.async_copy(table.at[src_v.at[qig]], rows_v.at[qg], gsems[qg])

      if b < IB - GA:
        gather_ahead()
      else:
        pl.when(t < NT - 1)(gather_ahead)
    return carry

  lax.fori_loop(0, NT, outer, 0)
  for p in range(SB):
    wait_scatter(p)
    wait_cnt(p)
  plsc.subcore_barrier()
  pltpu.sync_copy(acc_cnt.at[pl.ds(s * ROWS_PER_SUB, ROWS_PER_SUB)],
                  out_cnt.at[c, pl.ds(s * ROWS_PER_SUB, ROWS_PER_SUB)])
  pltpu.sync_copy(acc.at[pl.ds(s * ROWS_PER_SUB, ROWS_PER_SUB)],
                  out.at[c, pl.ds(s * ROWS_PER_SUB, ROWS_PER_SUB)])


def _tc1_body(acc0, acc1, cnt0, cnt1, x, wl1t, bl1, wr1t, h_ref, cnt_ref):
  agg = acc0[0] + acc1[0]
  cnt = jnp.maximum(cnt0[0] + cnt1[0], 1.0)
  mean = agg / cnt
  h = mean @ wl1t[...] + bl1[...] + x[...] @ wr1t[...]
  h_ref[...] = jnp.maximum(h, 0.0)
  cnt_ref[...] = cnt


def _tc2_body(acc0, acc1, cnt, h, wl2t, bl2, wr2t, wdec, blinv, uv_ref):
  agg = acc0[0] + acc1[0]
  mean = agg / cnt[...]
  z = mean @ wl2t[...] + bl2[...] + h[...] @ wr2t[...]
  uv_ref[...] = z @ wdec[...] + blinv[...]


@functools.partial(
    pl.kernel,
    out_type=jax.ShapeDtypeStruct((NW, PPW), jnp.float32),
    mesh=_MESH,
    scratch_types=[
        pltpu.VMEM((PCHUNK, DCH), jnp.int32),
        pltpu.VMEM((PCHUNK, DCH), jnp.int32),
        pltpu.VMEM((DCH,), jnp.float32),
        pltpu.VMEM((DCH,), jnp.float32),
        pltpu.VMEM((PPW,), jnp.float32),
        pltpu.SemaphoreType.DMA,
        pltpu.SemaphoreType.DMA,
    ],
)
def _decode(u_hbm, v_hbm, spw, dpw, out, sp_v, dp_v, us_v, vs_v, o_v, semu, semv):
  c = lax.axis_index("c")
  s = lax.axis_index("s")
  w = c * NS + s
  pltpu.sync_copy(spw.at[w], sp_v)
  pltpu.sync_copy(dpw.at[w], dp_v)

  def body(j, carry):
    cu = pltpu.async_copy(u_hbm.at[sp_v.at[j]], us_v, semu)
    cv = pltpu.async_copy(v_hbm.at[dp_v.at[j]], vs_v, semv)
    cu.wait()
    cv.wait()

    def sig(k, carry2):
      sl = pl.ds(k * LANES, LANES)
      logit = us_v[sl] + vs_v[sl]
      o_v[pl.ds(j * DCH + k * LANES, LANES)] = 1.0 / (1.0 + jnp.exp(-logit))
      return carry2

    lax.fori_loop(0, DCH // LANES, sig, 0)
    return carry

  lax.fori_loop(0, PCHUNK, body, 0)
  pltpu.sync_copy(o_v, out.at[w])


def _pad_worker_idx(idx, n_real, n_pad_total, pad_val):
  per_w = n_real // NW
  per_w_pad = n_pad_total // NW
  r = idx.reshape(NW, per_w)
  pad = jnp.full((NW, per_w_pad - per_w), pad_val, dtype=idx.dtype)
  return jnp.concatenate([r, pad], axis=1)


def kernel(x, edge_index, edge_pairs, Wl1, bl1, Wr1, Wl2, bl2, Wr2, Wlin, blin):
  f32 = jnp.float32
  src = _pad_worker_idx(edge_index[0], N_EDGES, EPW * NW, N_NODES)
  src = src.reshape(NW, NCHUNK, CH)
  dst = _pad_worker_idx(edge_index[1], N_EDGES, EPW * NW, N_NODES)
  dst = dst.reshape(NW, NCHUNK, CH)

  x_pad = jnp.concatenate([x, jnp.zeros((NPAD - N_NODES, D), f32)], axis=0)
  zrows = jnp.zeros((CH, D), f32)


  wl1t = Wl1.T
  wr1t = Wr1.T
  wl2t = Wl2.T
  wr2t = Wr2.T
  bl1r = bl1.reshape(1, D)
  bl2r = bl2.reshape(1, D)
  wdec = jnp.zeros((D, D), f32)
  wdec = wdec.at[:, 0].set(Wlin[0, :D]).at[:, 1].set(Wlin[0, D:])
  blinv = jnp.zeros((1, D), f32).at[0, 0].set(blin[0])

  sp = _pad_worker_idx(edge_pairs[0], N_PAIRS, PPW * NW, 0).reshape(NW, PCHUNK, DCH)
  dp = _pad_worker_idx(edge_pairs[1], N_PAIRS, PPW * NW, 0).reshape(NW, PCHUNK, DCH)

  acc1, cnt1 = _segsum(x_pad, src, dst, zrows)
  cnt1r = cnt1.reshape(NC, NPAD, 1)

  BLK = 1024
  grid = (NPAD // BLK,)
  full = lambda shape: pl.BlockSpec(shape, lambda i: tuple(0 for _ in shape))
  h, cntb = pl.pallas_call(
      _tc1_body,
      grid=grid,
      in_specs=[
          pl.BlockSpec((1, BLK, D), lambda i: (0, i, 0)),
          pl.BlockSpec((1, BLK, D), lambda i: (1, i, 0)),
          pl.BlockSpec((1, BLK, 1), lambda i: (0, i, 0)),
          pl.BlockSpec((1, BLK, 1), lambda i: (1, i, 0)),
          pl.BlockSpec((BLK, D), lambda i: (i, 0)),
          full((D, D)), full((1, D)), full((D, D)),
      ],
      out_specs=[
          pl.BlockSpec((BLK, D), lambda i: (i, 0)),
          pl.BlockSpec((BLK, 1), lambda i: (i, 0)),
      ],
      out_shape=[
          jax.ShapeDtypeStruct((NPAD, D), f32),
          jax.ShapeDtypeStruct((NPAD, 1), f32),
      ],
  )(acc1, acc1, cnt1r, cnt1r, x_pad, wl1t, bl1r, wr1t)

  acc2, _ = _segsum(h, src, dst, zrows)

  uv = pl.pallas_call(
      _tc2_body,
      grid=grid,
      in_specs=[
          pl.BlockSpec((1, BLK, D), lambda i: (0, i, 0)),
          pl.BlockSpec((1, BLK, D), lambda i: (1, i, 0)),
          pl.BlockSpec((BLK, 1), lambda i: (i, 0)),
          pl.BlockSpec((BLK, D), lambda i: (i, 0)),
          full((D, D)), full((1, D)), full((D, D)), full((D, D)), full((1, D)),
      ],
      out_specs=pl.BlockSpec((BLK, D), lambda i: (i, 0)),
      out_shape=jax.ShapeDtypeStruct((NPAD, D), f32),
  )(acc2, acc2, cntb, h, wl2t, bl2r, wr2t, wdec, blinv)

  u = uv[:, 0]
  v = uv[:, 1]

  probs = _decode(u, v, sp, dp)
  return probs.reshape(NW, PPW)[:, :N_PAIRS // NW].reshape(-1)


# speedup vs baseline: 4.3362x; 1.0022x over previous
import functools

import jax
import jax.numpy as jnp
from jax import lax
from jax.experimental import pallas as pl
from jax.experimental.pallas import tpu as pltpu
from jax.experimental.pallas import tpu_sc as plsc

N_NODES = 10000
N_EDGES = 320000
N_PAIRS = 100000
D = 128

NC = 2
NS = 16
NW = NC * NS
LANES = 16

NPAD = 10240
ROWS_PER_SUB = NPAD // NS
CH = 64
EPW = 10240
NCHUNK = EPW // CH
RB = 5
SB = 2
IB = 10
UNROLL = 10
NT = NCHUNK // UNROLL
GA = 3
IA = 6

DCH = 128
PPW = 3200
PCHUNK = PPW // DCH

_MESH = plsc.VectorSubcoreMesh(
    core_axis_name="c", subcore_axis_name="s", num_cores=NC, num_subcores=NS)


def _make_segsum(with_cnt):
  out_types = [jax.ShapeDtypeStruct((NC, NPAD, D), jnp.float32)]
  if with_cnt:
    out_types.append(jax.ShapeDtypeStruct((NC, NPAD), jnp.float32))

  @functools.partial(
      pl.kernel,
      out_type=out_types,
      mesh=_MESH,
      scratch_types=[
          pltpu.VMEM((IB, CH), jnp.int32),
          pltpu.VMEM((IB, CH), jnp.int32),
          pltpu.VMEM((RB, CH, D), jnp.float32),
          pltpu.VMEM((CH,), jnp.float32),
          pltpu.VMEM((DCH,), jnp.float32),
          pltpu.VMEM_SHARED((NPAD, D), jnp.float32),
          pltpu.VMEM_SHARED((NPAD,), jnp.float32),
          [pltpu.SemaphoreType.DMA] * IB,
          [pltpu.SemaphoreType.DMA] * RB,
          [pltpu.SemaphoreType.DMA] * SB,
          [pltpu.SemaphoreType.DMA] * SB,
      ],
  )
  def _segsum(table, srcw, dstw, zrows, *rest):
    if with_cnt:
      (out, out_cnt, src_v, dst_v, rows_v, ones_v, zc_v, acc, acc_cnt,
       isems, gsems, ssems, csems) = rest
    else:
      (out, src_v, dst_v, rows_v, ones_v, zc_v, acc, acc_cnt,
       isems, gsems, ssems, csems) = rest
    c = lax.axis_index("c")
    s = lax.axis_index("s")
    w = c * NS + s
    pltpu.sync_copy(zrows, rows_v.at[0])
    for k in range(ROWS_PER_SUB // CH):
      pltpu.sync_copy(rows_v.at[0], acc.at[pl.ds((s * (ROWS_PER_SUB // CH) + k) * CH, CH)])

    def zfill(i, carry):
      zc_v[pl.ds(i * LANES, LANES)] = jnp.zeros((LANES,), jnp.float32)
      return carry

    lax.fori_loop(0, DCH // LANES, zfill, 0)

    if with_cnt:
      def ofill(i, carry):
        ones_v[pl.ds(i * LANES, LANES)] = jnp.ones((LANES,), jnp.float32)
        return carry

      lax.fori_loop(0, CH // LANES, ofill, 0)
      for k in range(ROWS_PER_SUB // DCH):
        pltpu.sync_copy(zc_v, acc_cnt.at[pl.ds(s * ROWS_PER_SUB + k * DCH, DCH)])
    plsc.subcore_barrier()

    def wait_idx(qi):
      pltpu.make_async_copy(srcw.at[w, 0], src_v.at[qi], isems[qi]).wait()
      pltpu.make_async_copy(dstw.at[w, 0], dst_v.at[qi], isems[qi]).wait()

    def wait_gather(q):
      pltpu.make_async_copy(table.at[src_v.at[0]], rows_v.at[q], gsems[q]).wait()

    def wait_scatter(p):
      pltpu.make_async_copy(rows_v.at[0], acc.at[dst_v.at[0]], ssems[p]).wait()

    def wait_cnt(p):
      if with_cnt:
        pltpu.make_async_copy(ones_v, acc_cnt.at[dst_v.at[0]], csems[p]).wait()

    def prefetch_idx(jj, qslot):
      pltpu.async_copy(srcw.at[w, jj], src_v.at[qslot], isems[qslot])
      pltpu.async_copy(dstw.at[w, jj], dst_v.at[qslot], isems[qslot])

    for b in range(IA):
      prefetch_idx(b, b)
    for b in range(GA):
      wait_idx(b)
      pltpu.async_copy(table.at[src_v.at[b]], rows_v.at[b], gsems[b])

    def outer(t, carry):
      for b in range(UNROLL):
        j = t * UNROLL + b
        q = b % RB
        qg = (b + GA) % RB
        qi = b % IB
        qig = (b + GA) % IB
        qip = (b + IA) % IB
        p = b % SB

        def waits_a():
          wait_scatter(p)
          wait_cnt(p)

        if b < SB:
          pl.when(t > 0)(waits_a)
        else:
          waits_a()

        if b < IB - IA:
          prefetch_idx(j + IA, qip)
        else:
          pl.when(t < NT - 1)(lambda: prefetch_idx(j + IA, qip))

        wait_gather(q)
        pltpu.async_copy(rows_v.at[q], acc.at[dst_v.at[qi]], ssems[p], add=True)
        if with_cnt:
          pltpu.async_copy(ones_v, acc_cnt.at[dst_v.at[qi]], csems[p], add=True)

        def gather_ahead():
          wait_idx(qig)
          pltpu.async_copy(table.at[src_v.at[qig]], rows_v.at[qg], gsems[qg])

        if b < IB - GA:
          gather_ahead()
        else:
          pl.when(t < NT - 1)(gather_ahead)
      return carry

    lax.fori_loop(0, NT, outer, 0)
    for p in range(SB):
      wait_scatter(p)
      wait_cnt(p)
    plsc.subcore_barrier()
    if with_cnt:
      pltpu.sync_copy(acc_cnt.at[pl.ds(s * ROWS_PER_SUB, ROWS_PER_SUB)],
                      out_cnt.at[c, pl.ds(s * ROWS_PER_SUB, ROWS_PER_SUB)])
    pltpu.sync_copy(acc.at[pl.ds(s * ROWS_PER_SUB, ROWS_PER_SUB)],
                    out.at[c, pl.ds(s * ROWS_PER_SUB, ROWS_PER_SUB)])

  return _segsum


_segsum_cnt = _make_segsum(True)
_segsum_nc = _make_segsum(False)


def _tc1_body(acc0, acc1, cnt0, cnt1, x, wl1t, bl1, wr1t, h_ref, cnt_ref):
  agg = acc0[0] + acc1[0]
  cnt = jnp.maximum(cnt0[0] + cnt1[0], 1.0)
  mean = agg / cnt
  h = mean @ wl1t[...] + bl1[...] + x[...] @ wr1t[...]
  h_ref[...] = jnp.maximum(h, 0.0)
  cnt_ref[...] = cnt


def _tc2_body(acc0, acc1, cnt, h, wl2t, bl2, wr2t, wdec, blinv, uv_ref):
  agg = acc0[0] + acc1[0]
  mean = agg / cnt[...]
  z = mean @ wl2t[...] + bl2[...] + h[...] @ wr2t[...]
  uv_ref[...] = z @ wdec[...] + blinv[...]


@functools.partial(
    pl.kernel,
    out_type=jax.ShapeDtypeStruct((NW, PPW), jnp.float32),
    mesh=_MESH,
    scratch_types=[
        pltpu.VMEM((PCHUNK, DCH), jnp.int32),
        pltpu.VMEM((PCHUNK, DCH), jnp.int32),
        pltpu.VMEM((DCH,), jnp.float32),
        pltpu.VMEM((DCH,), jnp.float32),
        pltpu.VMEM((PPW,), jnp.float32),
        pltpu.SemaphoreType.DMA,
        pltpu.SemaphoreType.DMA,
    ],
)
def _decode(uvf, spw, dpw, out, sp_v, dp_v, us_v, vs_v, o_v, semu, semv):
  c = lax.axis_index("c")
  s = lax.axis_index("s")
  w = c * NS + s
  pltpu.sync_copy(spw.at[w], sp_v)
  pltpu.sync_copy(dpw.at[w], dp_v)

  def body(j, carry):
    cu = pltpu.async_copy(uvf.at[sp_v.at[j]], us_v, semu)
    cv = pltpu.async_copy(uvf.at[dp_v.at[j]], vs_v, semv)
    cu.wait()
    cv.wait()

    def sig(k, carry2):
      sl = pl.ds(k * LANES, LANES)
      logit = us_v[sl] + vs_v[sl]
      o_v[pl.ds(j * DCH + k * LANES, LANES)] = 1.0 / (1.0 + jnp.exp(-logit))
      return carry2

    lax.fori_loop(0, DCH // LANES, sig, 0)
    return carry

  lax.fori_loop(0, PCHUNK, body, 0)
  pltpu.sync_copy(o_v, out.at[w])


def _pad_worker_idx(idx, n_real, n_pad_total, pad_val):
  per_w = n_real // NW
  per_w_pad = n_pad_total // NW
  r = idx.reshape(NW, per_w)
  pad = jnp.full((NW, per_w_pad - per_w), pad_val, dtype=idx.dtype)
  return jnp.concatenate([r, pad], axis=1)


def kernel(x, edge_index, edge_pairs, Wl1, bl1, Wr1, Wl2, bl2, Wr2, Wlin, blin):
  f32 = jnp.float32
  src = _pad_worker_idx(edge_index[0], N_EDGES, EPW * NW, N_NODES)
  src = src.reshape(NW, NCHUNK, CH)
  dst = _pad_worker_idx(edge_index[1], N_EDGES, EPW * NW, N_NODES)
  dst = dst.reshape(NW, NCHUNK, CH)

  x_pad = jnp.concatenate([x, jnp.zeros((NPAD - N_NODES, D), f32)], axis=0)
  zrows = jnp.zeros((CH, D), f32)


  wl1t = Wl1.T
  wr1t = Wr1.T
  wl2t = Wl2.T
  wr2t = Wr2.T
  bl1r = bl1.reshape(1, D)
  bl2r = bl2.reshape(1, D)
  wdec = jnp.zeros((D, D), f32)
  wdec = wdec.at[:, 0].set(Wlin[0, :D]).at[:, 1].set(Wlin[0, D:])
  blinv = jnp.zeros((1, D), f32).at[0, 0].set(blin[0])

  sp = _pad_worker_idx(edge_pairs[0] * D, N_PAIRS, PPW * NW, 0)
  sp = sp.reshape(NW, PCHUNK, DCH)
  dp = _pad_worker_idx(edge_pairs[1] * D + 1, N_PAIRS, PPW * NW, 1)
  dp = dp.reshape(NW, PCHUNK, DCH)

  acc1, cnt1 = _segsum_cnt(x_pad, src, dst, zrows)
  cnt1r = cnt1.reshape(NC, NPAD, 1)

  BLK = 1024
  grid = (NPAD // BLK,)
  full = lambda shape: pl.BlockSpec(shape, lambda i: tuple(0 for _ in shape))
  h, cntb = pl.pallas_call(
      _tc1_body,
      grid=grid,
      in_specs=[
          pl.BlockSpec((1, BLK, D), lambda i: (0, i, 0)),
          pl.BlockSpec((1, BLK, D), lambda i: (1, i, 0)),
          pl.BlockSpec((1, BLK, 1), lambda i: (0, i, 0)),
          pl.BlockSpec((1, BLK, 1), lambda i: (1, i, 0)),
          pl.BlockSpec((BLK, D), lambda i: (i, 0)),
          full((D, D)), full((1, D)), full((D, D)),
      ],
      out_specs=[
          pl.BlockSpec((BLK, D), lambda i: (i, 0)),
          pl.BlockSpec((BLK, 1), lambda i: (i, 0)),
      ],
      out_shape=[
          jax.ShapeDtypeStruct((NPAD, D), f32),
          jax.ShapeDtypeStruct((NPAD, 1), f32),
      ],
  )(acc1, acc1, cnt1r, cnt1r, x_pad, wl1t, bl1r, wr1t)

  (acc2,) = _segsum_nc(h, src, dst, zrows)

  uv = pl.pallas_call(
      _tc2_body,
      grid=grid,
      in_specs=[
          pl.BlockSpec((1, BLK, D), lambda i: (0, i, 0)),
          pl.BlockSpec((1, BLK, D), lambda i: (1, i, 0)),
          pl.BlockSpec((BLK, 1), lambda i: (i, 0)),
          pl.BlockSpec((BLK, D), lambda i: (i, 0)),
          full((D, D)), full((1, D)), full((D, D)), full((D, D)), full((1, D)),
      ],
      out_specs=pl.BlockSpec((BLK, D), lambda i: (i, 0)),
      out_shape=jax.ShapeDtypeStruct((NPAD, D), f32),
  )(acc2, acc2, cntb, h, wl2t, bl2r, wr2t, wdec, blinv)

  probs = _decode(uv.reshape(-1), sp, dp)
  return probs.reshape(NW, PPW)[:, :N_PAIRS // NW].reshape(-1)
